# Initial kernel scaffold; baseline (speedup 1.0000x reference)
import jax
import jax.numpy as jnp
from jax.experimental import pallas as pl


def kernel(state, action, edge_index, edge_attr, batch, W_in, b_in, W_tag, b_tag, W_out, b_out):
    raise NotImplementedError("write your pallas kernel here")


# speedup vs baseline: 11.2301x; 11.2301x over previous
import jax
import jax.numpy as jnp
from jax import lax
from jax.experimental import pallas as pl
from jax.experimental.pallas import tpu as pltpu
from jax.experimental.pallas import tpu_sc as plsc
from jax.scipy.linalg import block_diag

N = 100000
E = 1600000
G = 64
C = 32
H = 16
E2 = 1638400
ER = E2 // 128
NPAD = 102400
TS = 6400
NP8 = NPAD // 8
PR = 400
NB = NP8 // PR

_mesh = plsc.VectorSubcoreMesh(core_axis_name="c", subcore_axis_name="s")


def _deg_body(dst_h, ea_h, out_h, deg_sp, dstv, eav, zbuf):
    c = lax.axis_index("c")
    s = lax.axis_index("s")

    def _z(i, _):
        zbuf[pl.ds(i * 16, 16)] = jnp.zeros((16,), jnp.float32)
        return 0
    lax.fori_loop(0, TS // 16, _z, 0)
    pltpu.sync_copy(zbuf, deg_sp.at[pl.ds(s * TS, TS)])
    plsc.subcore_barrier()

    base = (c * 16 + s) * 400

    def _chunk(lc, _):
        row = base + lc * 16
        pltpu.sync_copy(dst_h.at[pl.ds(row, 16)], dstv)
        pltpu.sync_copy(ea_h.at[pl.ds(row, 16)], eav)

        def _j(j, _):
            pltpu.sync_copy(eav.at[j], deg_sp.at[dstv.at[j]], add=True)
            return 0
        lax.fori_loop(0, 16, _j, 0)
        return 0
    lax.fori_loop(0, 25, _chunk, 0)

    plsc.subcore_barrier()
    pltpu.sync_copy(deg_sp.at[pl.ds(s * TS, TS)],
                    out_h.at[pl.ds(c * NPAD + s * TS, TS)])


def _deg(dst2, ea2):
    f = pl.kernel(
        _deg_body,
        out_type=jax.ShapeDtypeStruct((2 * NPAD,), jnp.float32),
        mesh=_mesh,
        compiler_params=pltpu.CompilerParams(use_tc_tiling_on_sc=False),
        scratch_types=[
            pltpu.VMEM_SHARED((NPAD,), jnp.float32),
            pltpu.VMEM((16, 128), jnp.int32),
            pltpu.VMEM((16, 128), jnp.float32),
            pltpu.VMEM((TS,), jnp.float32),
        ],
    )
    return f(dst2, ea2)


def _norm_body(src_h, dst_h, ea_h, dinv_h, out_h, srcv, dstv, eav, gs, gd, nv,
               sem):
    c = lax.axis_index("c")
    s = lax.axis_index("s")
    base = (s * 2 + c) * 400

    def _chunk(lc, _):
        row = base + lc * 16
        pltpu.sync_copy(src_h.at[pl.ds(row, 16)], srcv)
        pltpu.sync_copy(dst_h.at[pl.ds(row, 16)], dstv)
        pltpu.sync_copy(ea_h.at[pl.ds(row, 16)], eav)

        def _fire(j, _):
            pltpu.async_copy(dinv_h.at[srcv.at[j]], gs.at[j], sem)
            pltpu.async_copy(dinv_h.at[dstv.at[j]], gd.at[j], sem)
            return 0
        lax.fori_loop(0, 16, _fire, 0)

        def _drain(j, _):
            pltpu.make_async_copy(dinv_h.at[srcv.at[j]], gs.at[j], sem).wait()
            pltpu.make_async_copy(dinv_h.at[dstv.at[j]], gd.at[j], sem).wait()
            return 0
        lax.fori_loop(0, 16, _drain, 0)

        def _mul(t, _):
            r = t // 8
            k = t % 8
            sl = pl.ds(k * 16, 16)
            nv[r, sl] = gs[r, sl] * eav[r, sl] * gd[r, sl]
            return 0
        lax.fori_loop(0, 128, _mul, 0)
        pltpu.sync_copy(nv, out_h.at[pl.ds(row, 16)])
        return 0
    lax.fori_loop(0, 25, _chunk, 0)


def _norm(src2, dst2, ea2, dinv):
    f = pl.kernel(
        _norm_body,
        out_type=jax.ShapeDtypeStruct((ER, 128), jnp.float32),
        mesh=_mesh,
        compiler_params=pltpu.CompilerParams(use_tc_tiling_on_sc=False),
        scratch_types=[
            pltpu.VMEM((16, 128), jnp.int32),
            pltpu.VMEM((16, 128), jnp.int32),
            pltpu.VMEM((16, 128), jnp.float32),
            pltpu.VMEM((16, 128), jnp.float32),
            pltpu.VMEM((16, 128), jnp.float32),
            pltpu.VMEM((16, 128), jnp.float32),
            pltpu.SemaphoreType.DMA,
        ],
    )
    return f(src2, dst2, ea2, dinv)


def _mp_body(src_h, dst_h, nrm_h, x_h, o1, o2, o3, o4,
             acc_sp, rows, srcv, idxv, nrmv, dstv, zbuf, sem):
    c = lax.axis_index("c")
    s = lax.axis_index("s")
    cN = c * NPAD

    def _z(i, _):
        zbuf[i, :] = jnp.zeros((16,), jnp.float32)
        return 0
    lax.fori_loop(0, 128, _z, 0)

    outs = [o1, o2, o3, o4]
    for r in range(4):
        tbl = x_h if r == 0 else outs[r - 1]
        out = outs[r]

        def _zero(q, _):
            pltpu.sync_copy(zbuf, acc_sp.at[pl.ds(s * TS + q * 128, 128)])
            return 0
        lax.fori_loop(0, TS // 128, _zero, 0)
        plsc.subcore_barrier()

        def _chunk(lc, _):
            row = s * 800 + lc * 8
            pltpu.sync_copy(src_h.at[pl.ds(row, 8)], srcv)
            pltpu.sync_copy(nrm_h.at[pl.ds(row, 8)], nrmv)
            pltpu.sync_copy(dst_h.at[pl.ds(row, 8)], dstv)

            def _idx(t, _):
                r2 = t // 8
                sl = pl.ds((t % 8) * 16, 16)
                idxv[r2, sl] = srcv[r2, sl] + cN
                return 0
            lax.fori_loop(0, 64, _idx, 0)

            def _fire(j, _):
                pltpu.async_copy(tbl.at[idxv.at[j]], rows.at[j], sem)
                return 0
            lax.fori_loop(0, 8, _fire, 0)

            def _drain(j, _):
                pltpu.make_async_copy(tbl.at[idxv.at[j]], rows.at[j],
                                      sem).wait()
                return 0
            lax.fori_loop(0, 8, _drain, 0)

            def _scale_scatter(j, _):
                for ib in range(8):
                    nv16 = nrmv[j, pl.ds(ib * 16, 16)]
                    for e in range(16):
                        i = ib * 16 + e
                        nb = jnp.full((16,), nv16[e], jnp.float32)
                        rows[j, i, :] = rows[j, i, :] * nb
                pltpu.sync_copy(rows.at[j], acc_sp.at[dstv.at[j]], add=True)
                return 0
            lax.fori_loop(0, 8, _scale_scatter, 0)
            return 0
        lax.fori_loop(0, 100, _chunk, 0)

        plsc.subcore_barrier()
        pltpu.sync_copy(acc_sp.at[pl.ds(s * TS, TS)],
                        out.at[pl.ds(cN + s * TS, TS)])
        plsc.subcore_barrier()


def _mp(src2, dst2, nrm2, x2n):
    f = pl.kernel(
        _mp_body,
        out_type=tuple(jax.ShapeDtypeStruct((2 * NPAD, H), jnp.float32)
                       for _ in range(4)),
        mesh=_mesh,
        compiler_params=pltpu.CompilerParams(use_tc_tiling_on_sc=False),
        scratch_types=[
            pltpu.VMEM_SHARED((NPAD, H), jnp.float32),
            pltpu.VMEM((8, 128, H), jnp.float32),
            pltpu.VMEM((8, 128), jnp.int32),
            pltpu.VMEM((8, 128), jnp.int32),
            pltpu.VMEM((8, 128), jnp.float32),
            pltpu.VMEM((8, 128), jnp.int32),
            pltpu.VMEM((128, 16), jnp.float32),
            pltpu.SemaphoreType.DMA,
        ],
    )
    return f(src2, dst2, nrm2, x2n)


def _leaky(z):
    return jnp.where(z > 0, z, 0.01 * z)


def _readin_body(s_ref, a_ref, ws_ref, wa_ref, b_ref, o_ref):
    z = (jnp.dot(s_ref[...], ws_ref[...], preferred_element_type=jnp.float32)
         + jnp.dot(a_ref[...], wa_ref[...], preferred_element_type=jnp.float32)
         + b_ref[0:1, :])
    z = _leaky(z)
    o_ref[...] = jnp.stack([z[:, :128], z[:, 128:]], axis=0)


def _readin(state_p, action_p, WS, WA, b2):
    return pl.pallas_call(
        _readin_body,
        grid=(NB,),
        in_specs=[
            pl.BlockSpec((PR, 960), lambda i: (i, 0)),
            pl.BlockSpec((PR, 64), lambda i: (i, 0)),
            pl.BlockSpec((960, 256), lambda i: (0, 0)),
            pl.BlockSpec((64, 256), lambda i: (0, 0)),
            pl.BlockSpec((8, 256), lambda i: (0, 0)),
        ],
        out_specs=pl.BlockSpec((2, PR, 128), lambda i: (0, i, 0)),
        out_shape=jax.ShapeDtypeStruct((2, NP8, 128), jnp.float32),
    )(state_p, action_p, WS, WA, b2)


def _dinv_body(d_ref, o_ref):
    d = d_ref[0:8, :] + d_ref[8:16, :]
    o_ref[...] = jnp.where(d > 0, lax.rsqrt(jnp.maximum(d, 1e-12)), 0.0)


def _dinv(degp16):
    return pl.pallas_call(
        _dinv_body,
        grid=(1,),
        in_specs=[pl.BlockSpec((16, NPAD // 8), lambda i: (0, 0))],
        out_specs=pl.BlockSpec((8, NPAD // 8), lambda i: (0, 0)),
        out_shape=jax.ShapeDtypeStruct((8, NPAD // 8), jnp.float32),
    )(degp16)


def _cat10(x0, x1, x2, x3, x4):
    return jnp.concatenate(
        [x0[0], x1[0], x2[0], x3[0], x4[0],
         x0[1], x1[1], x2[1], x3[1], x4[1]], axis=1)


def _tapmm_body(x0, x1, x2, x3, x4, wt_ref, b_ref, o_ref):
    xc = _cat10(x0[...], x1[...], x2[...], x3[...], x4[...])
    z = jnp.dot(xc, wt_ref[...], preferred_element_type=jnp.float32) \
        + b_ref[0:1, :]
    z = _leaky(z)
    o_ref[...] = jnp.stack([z[:, :128], z[:, 128:]], axis=0)


def _tapmm(xs, WT, b2):
    xspec = pl.BlockSpec((2, PR, 128), lambda i: (0, i, 0))
    return pl.pallas_call(
        _tapmm_body,
        grid=(NB,),
        in_specs=[xspec] * 5 + [
            pl.BlockSpec((1280, 256), lambda i: (0, 0)),
            pl.BlockSpec((8, 256), lambda i: (0, 0)),
        ],
        out_specs=pl.BlockSpec((2, PR, 128), lambda i: (0, i, 0)),
        out_shape=jax.ShapeDtypeStruct((2, NP8, 128), jnp.float32),
    )(*xs, WT, b2)


def _readout_body(x0, x1, x2, x3, x4, wt_ref, b_ref, wo_ref, bo_ref, bat_ref,
                  o_ref, sums, cnts):
    i = pl.program_id(0)
    xc = _cat10(x0[...], x1[...], x2[...], x3[...], x4[...])
    z = jnp.dot(xc, wt_ref[...], preferred_element_type=jnp.float32) \
        + b_ref[0:1, :]
    z = _leaky(z)
    y = jnp.dot(z, wo_ref[...], preferred_element_type=jnp.float32) \
        + bo_ref[0, 0]
    gids = lax.broadcasted_iota(jnp.int32, (1, G), 1)

    @pl.when(i == 0)
    def _():
        sums[...] = jnp.zeros((G, 1), jnp.float32)
        cnts[...] = jnp.zeros((G, 1), jnp.float32)

    bsum = jnp.zeros((G,), jnp.float32)
    bcnt = jnp.zeros((G,), jnp.float32)
    for c2 in range(8):
        m = (bat_ref[0, :, c2:c2 + 1] == gids).astype(jnp.float32)
        bsum = bsum + jnp.sum(m * y[:, c2:c2 + 1], axis=0)
        bcnt = bcnt + jnp.sum(m, axis=0)
    sums[...] += bsum.reshape(G, 1)
    cnts[...] += bcnt.reshape(G, 1)

    @pl.when(i == NB - 1)
    def _():
        o_ref[...] = sums[...] / jnp.maximum(cnts[...], 1.0)


def _readout(xs, WT, b2, WoP, bo8, batch_p):
    xspec = pl.BlockSpec((2, PR, 128), lambda i: (0, i, 0))
    return pl.pallas_call(
        _readout_body,
        grid=(NB,),
        in_specs=[xspec] * 5 + [
            pl.BlockSpec((1280, 256), lambda i: (0, 0)),
            pl.BlockSpec((8, 256), lambda i: (0, 0)),
            pl.BlockSpec((256, 8), lambda i: (0, 0)),
            pl.BlockSpec((8, 128), lambda i: (0, 0)),
            pl.BlockSpec((1, PR, 8), lambda i: (i, 0, 0)),
        ],
        out_specs=pl.BlockSpec((G, 1), lambda i: (0, 0)),
        out_shape=jax.ShapeDtypeStruct((G, 1), jnp.float32),
        scratch_shapes=[
            pltpu.VMEM((G, 1), jnp.float32),
            pltpu.VMEM((G, 1), jnp.float32),
        ],
    )(*xs, WT, b2, WoP, bo8, batch_p)


def _pack_w(W):
    lo = block_diag(*([W[:, :H]] * 8))
    hi = block_diag(*([W[:, H:]] * 8))
    return jnp.concatenate([lo, hi], axis=1)


def _pack_b(b):
    return jnp.broadcast_to(
        jnp.concatenate([jnp.tile(b[:H], 8), jnp.tile(b[H:], 8)]).reshape(
            1, 256), (8, 256))


def _pack_wt(Wl):
    parts = ([_pack_w(Wl[k, :H, :]) for k in range(5)]
             + [_pack_w(Wl[k, H:, :]) for k in range(5)])
    return jnp.concatenate(parts, axis=0)


def kernel(state, action, edge_index, edge_attr, batch, W_in, b_in, W_tag,
           b_tag, W_out, b_out):
    pad = E2 - E
    src = edge_index[0].astype(jnp.int32)
    dst = edge_index[1].astype(jnp.int32)
    src2 = jnp.concatenate([src, jnp.zeros((pad,), jnp.int32)]).reshape(ER, 128)
    dst2 = jnp.concatenate(
        [dst, (jnp.arange(pad, dtype=jnp.int32) % N)]).reshape(ER, 128)
    ea2 = jnp.concatenate(
        [edge_attr.astype(jnp.float32),
         jnp.zeros((pad,), jnp.float32)]).reshape(ER, 128)

    npd = NPAD - N
    state_p = jnp.concatenate(
        [state, jnp.zeros((npd, 120), jnp.float32)]).reshape(NP8, 960)
    action_p = jnp.concatenate(
        [action, jnp.zeros((npd, 8), jnp.float32)]).reshape(NP8, 64)
    WS = _pack_w(W_in[:120])
    WA = _pack_w(W_in[120:])
    WT0 = _pack_wt(W_tag[0])
    WT1 = _pack_wt(W_tag[1])
    WoP = jnp.concatenate([block_diag(*([W_out[:H]] * 8)),
                           block_diag(*([W_out[H:]] * 8))], axis=0)
    bo8 = jnp.broadcast_to(b_out.reshape(1, 1), (8, 128))
    batch_p = jnp.concatenate(
        [batch.astype(jnp.int32), jnp.full((npd,), G, jnp.int32)]).reshape(
            NB, PR, 8)

    x = _readin(state_p, action_p, WS, WA, _pack_b(b_in))
    degp = _deg(dst2, ea2)
    dinv = _dinv(degp.reshape(16, NPAD // 8)).reshape(NPAD)
    nrm2 = _norm(src2, dst2, ea2, dinv)

    x_flat = x.reshape(2 * NPAD, H)
    t1 = _mp(src2, dst2, nrm2, x_flat)
    xs0 = [x] + [t.reshape(2, NP8, 128) for t in t1]
    x1 = _tapmm(xs0, WT0, _pack_b(b_tag[0]))

    t2 = _mp(src2, dst2, nrm2, x1.reshape(2 * NPAD, H))
    xs1 = [x1] + [t.reshape(2, NP8, 128) for t in t2]
    return _readout(xs1, WT1, _pack_b(b_tag[1]), WoP, bo8, batch_p)
